# baseline (speedup 1.0000x reference)
import functools

import jax
import jax.numpy as jnp
from jax import lax
from jax.experimental import pallas as pl
from jax.experimental.pallas import tpu as pltpu
from jax.experimental.pallas import tpu_sc as plsc

_SAMPLE_N = 16384
_FEAT = 128
_NC = 2
_NS = 16
_NW = _NC * _NS
_ROWS_PER_W = _SAMPLE_N // _NW


def _make_sc_kernel():
    mesh = plsc.VectorSubcoreMesh(core_axis_name="c", subcore_axis_name="s")

    @functools.partial(
        pl.kernel,
        mesh=mesh,
        out_type=jax.ShapeDtypeStruct((_SAMPLE_N, _FEAT), jnp.float32),
        scratch_types=[
            pltpu.VMEM((_ROWS_PER_W, _FEAT), jnp.float32),
            pltpu.SemaphoreType.DMA,
        ],
    )
    def k(ds_hbm, out_hbm, buf, sem):
        wid = lax.axis_index("s") * _NC + lax.axis_index("c")
        base = wid * _ROWS_PER_W
        pltpu.async_copy(ds_hbm.at[pl.ds(base, _ROWS_PER_W), :], buf, sem).wait()
        pltpu.async_copy(buf, out_hbm.at[pl.ds(base, _ROWS_PER_W), :], sem).wait()

    return k


_sc_kernel = _make_sc_kernel()


def kernel(dataset):
    return _sc_kernel(dataset)
